# baseline (speedup 1.0000x reference)
import functools

import jax
import jax.numpy as jnp
from jax import lax
from jax.experimental import pallas as pl
from jax.experimental.pallas import tpu as pltpu
from jax.experimental.pallas import tpu_sc as plsc

N = 50000
D = 256
NUM_GRAPHS = 64
CHUNK = 120
LANES = 16
NCOLG = D // LANES
N_PAD = 50048
UNROLL = 8


def _scalar_at(ref, idx):
    v = ref[0, pl.ds(idx, LANES)]
    return v[0]


def _starts_body(batch_ref, starts_ref):
    b = batch_ref[...]
    lane = lax.broadcasted_iota(jnp.int32, (1, 128), 1)
    acc = jnp.zeros((1, 128), jnp.int32)
    for g in range(NUM_GRAPHS + 1):
        c = jnp.sum((b < g).astype(jnp.int32))
        acc = acc + jnp.where(lane == g, c, 0)
    starts_ref[...] = acc


def _starts(batch):
    return pl.pallas_call(
        _starts_body,
        out_shape=jax.ShapeDtypeStruct((1, 128), jnp.int32),
    )(batch)


def _win(lo_al, c):
    return pl.multiple_of(jnp.minimum(lo_al + c * CHUNK, N - CHUNK), 8)


def _sc_pool_body(h_hbm, starts_hbm, sum_hbm, max_hbm, cnt_hbm,
                  starts_v, buf0, buf1, buf2, buf3, sum_st, max_st, cnt_st,
                  sem0, sem1, sem2, sem3, osem):
    wid = lax.axis_index("s") * 2 + lax.axis_index("c")
    pltpu.sync_copy(starts_hbm, starts_v)
    bufs = ((buf0, buf1), (buf2, buf3))
    sems = ((sem0, sem1), (sem2, sem3))

    params = []
    for gi in range(2):
        g = wid * 2 + gi
        lo = _scalar_at(starts_v, g)
        hi = _scalar_at(starts_v, g + 1)
        lo_al = (lo // 8) * 8
        num_chunks = jnp.maximum((hi - lo_al + CHUNK - 1) // CHUNK, 1)
        m = ((num_chunks + 1) // 2) * 2
        params.append((g, lo, hi, lo_al, m))
        for parity in range(2):
            pltpu.async_copy(h_hbm.at[pl.ds(_win(lo_al, parity), CHUNK)],
                             bufs[gi][parity], sems[gi][parity])

    for gi in range(2):
        g, lo, hi, lo_al, m = params[gi]
        n = hi - lo

        def pair_body(c2, carry, lo=lo, hi=hi, lo_al=lo_al, m=m, gi=gi):
            for parity in range(2):
                c = c2 * 2 + parity
                bf = bufs[gi][parity]
                w = _win(lo_al, c)
                pltpu.make_async_copy(
                    h_hbm.at[pl.ds(w, CHUNK)], bf, sems[gi][parity]).wait()
                base = lo_al + c * CHUNK
                r_start = jnp.clip(jnp.maximum(lo, base) - w, 0, CHUNK)
                r_end = jnp.clip(jnp.minimum(hi, base + CHUNK) - w, 0, CHUNK)
                r_end = jnp.maximum(r_start, r_end)
                nu = (r_end - r_start) // UNROLL

                def rows_at(r0, k, carry2, bf=bf):
                    sums2, maxs2 = carry2
                    new_s = list(sums2)
                    new_m = list(maxs2)
                    for j in range(NCOLG):
                        for rr in range(k):
                            v = bf[r0 + rr, pl.ds(j * LANES, LANES)]
                            new_s[j] = new_s[j] + v
                            new_m[j] = jnp.maximum(new_m[j], v)
                    return (tuple(new_s), tuple(new_m))

                def blk_body(i, carry2, rs=r_start):
                    return rows_at(rs + i * UNROLL, UNROLL, carry2)

                def row_body(r, carry2):
                    return rows_at(r, 1, carry2)

                carry = lax.fori_loop(0, nu, blk_body, carry)
                carry = lax.fori_loop(r_start + nu * UNROLL, r_end, row_body,
                                      carry)

                @pl.when(c + 2 < m)
                def _(c=c, bf=bf, parity=parity, lo_al=lo_al, gi=gi):
                    pltpu.async_copy(h_hbm.at[pl.ds(_win(lo_al, c + 2), CHUNK)],
                                     bf, sems[gi][parity])
            return carry

        init = (tuple(jnp.zeros((LANES,), jnp.float32) for _ in range(NCOLG)),
                tuple(jnp.full((LANES,), -jnp.inf, jnp.float32)
                      for _ in range(NCOLG)))
        sums, maxs = lax.fori_loop(0, m // 2, pair_body, init)

        for j in range(NCOLG):
            sum_st[gi, pl.ds(j * LANES, LANES)] = sums[j]
            max_st[gi, pl.ds(j * LANES, LANES)] = maxs[j]
        cnt_st[gi, pl.ds(0, LANES)] = (jnp.full((LANES,), 1.0, jnp.float32)
                                       * n.astype(jnp.float32))
        pltpu.async_copy(sum_st.at[pl.ds(gi, 1)], sum_hbm.at[pl.ds(g, 1)], osem)
        pltpu.async_copy(max_st.at[pl.ds(gi, 1)], max_hbm.at[pl.ds(g, 1)], osem)
        pltpu.async_copy(cnt_st.at[pl.ds(gi, 1)], cnt_hbm.at[pl.ds(g, 1)], osem)

    for gi in range(2):
        g = params[gi][0]
        pltpu.make_async_copy(sum_st.at[pl.ds(gi, 1)],
                              sum_hbm.at[pl.ds(g, 1)], osem).wait()
        pltpu.make_async_copy(max_st.at[pl.ds(gi, 1)],
                              max_hbm.at[pl.ds(g, 1)], osem).wait()
        pltpu.make_async_copy(cnt_st.at[pl.ds(gi, 1)],
                              cnt_hbm.at[pl.ds(g, 1)], osem).wait()


def _sc_pool(h, starts128):
    mesh = plsc.VectorSubcoreMesh(core_axis_name="c", subcore_axis_name="s")
    f = pl.kernel(
        _sc_pool_body,
        mesh=mesh,
        out_type=[
            jax.ShapeDtypeStruct((NUM_GRAPHS, D), jnp.float32),
            jax.ShapeDtypeStruct((NUM_GRAPHS, D), jnp.float32),
            jax.ShapeDtypeStruct((NUM_GRAPHS, LANES), jnp.float32),
        ],
        scratch_types=[
            pltpu.VMEM((1, 128), jnp.int32),
            pltpu.VMEM((CHUNK, D), jnp.float32),
            pltpu.VMEM((CHUNK, D), jnp.float32),
            pltpu.VMEM((CHUNK, D), jnp.float32),
            pltpu.VMEM((CHUNK, D), jnp.float32),
            pltpu.VMEM((2, D), jnp.float32),
            pltpu.VMEM((2, D), jnp.float32),
            pltpu.VMEM((2, LANES), jnp.float32),
            pltpu.SemaphoreType.DMA,
            pltpu.SemaphoreType.DMA,
            pltpu.SemaphoreType.DMA,
            pltpu.SemaphoreType.DMA,
            pltpu.SemaphoreType.DMA,
        ],
    )
    return f(h, starts128)


def _mlp_body(cnt_ref, sum_ref, max_ref, w1_ref, b1_ref, w2_ref, b2_ref,
              out_ref):
    cnt = jnp.maximum(cnt_ref[:, 0:1], 1.0)
    h_sum = sum_ref[...]
    h_max = max_ref[...]
    h_mean = h_sum / cnt
    x = jnp.concatenate([h_sum, h_mean, h_max], axis=1)
    y = lax.dot_general(x, w1_ref[...], (((1,), (1,)), ((), ())),
                        preferred_element_type=jnp.float32) + b1_ref[...]
    y = 0.5 * y * (1.0 + lax.erf(y * 0.7071067811865476))
    out_ref[...] = lax.dot_general(y, w2_ref[...], (((1,), (1,)), ((), ())),
                                   preferred_element_type=jnp.float32) + b2_ref[...]


def _mlp(cnt, h_sum, h_max, W1, b1, W2, b2):
    return pl.pallas_call(
        _mlp_body,
        out_shape=jax.ShapeDtypeStruct((NUM_GRAPHS, D), jnp.float32),
    )(cnt, h_sum, h_max, W1, b1.reshape(1, D), W2, b2.reshape(1, D))


def kernel(h, batch, W1, b1, W2, b2):
    starts128 = _starts(batch)
    h_sum, h_max, cnt = _sc_pool(h, starts128)
    return _mlp(cnt, h_sum, h_max, W1, b1, W2, b2)


# speedup vs baseline: 1.1575x; 1.1575x over previous
import functools

import jax
import jax.numpy as jnp
from jax import lax
from jax.experimental import pallas as pl
from jax.experimental.pallas import tpu as pltpu
from jax.experimental.pallas import tpu_sc as plsc

N = 50000
D = 256
NUM_GRAPHS = 64
CHUNK = 120
LANES = 16
NCOLG = D // LANES
N_PAD = 50048
UNROLL = 8


def _scalar_at(ref, idx):
    v = ref[0, pl.ds(idx, LANES)]
    return v[0]


def _starts_body(batch_ref, starts_ref):
    b = batch_ref[...]
    lane = lax.broadcasted_iota(jnp.int32, (1, 128), 1)
    acc = jnp.zeros((1, 128), jnp.int32)
    for g in range(NUM_GRAPHS + 1):
        c = jnp.sum((b < g).astype(jnp.int32))
        acc = acc + jnp.where(lane == g, c, 0)
    starts_ref[...] = acc


def _starts(batch):
    return pl.pallas_call(
        _starts_body,
        out_shape=jax.ShapeDtypeStruct((1, 128), jnp.int32),
    )(batch.reshape(400, 125))


def _win(lo_al, c):
    return pl.multiple_of(jnp.minimum(lo_al + c * CHUNK, N - CHUNK), 8)


def _sc_pool_body(h_hbm, starts_hbm, sum_hbm, max_hbm, cnt_hbm,
                  starts_v, buf0, buf1, buf2, buf3, sum_st, max_st, cnt_st,
                  sem0, sem1, sem2, sem3, osem):
    wid = lax.axis_index("s") * 2 + lax.axis_index("c")
    pltpu.sync_copy(starts_hbm, starts_v)
    bufs = ((buf0, buf1), (buf2, buf3))
    sems = ((sem0, sem1), (sem2, sem3))

    params = []
    for gi in range(2):
        g = wid * 2 + gi
        lo = _scalar_at(starts_v, g)
        hi = _scalar_at(starts_v, g + 1)
        lo_al = (lo // 8) * 8
        num_chunks = jnp.maximum((hi - lo_al + CHUNK - 1) // CHUNK, 1)
        m = ((num_chunks + 1) // 2) * 2
        params.append((g, lo, hi, lo_al, m))
        for parity in range(2):
            pltpu.async_copy(h_hbm.at[pl.ds(_win(lo_al, parity), CHUNK)],
                             bufs[gi][parity], sems[gi][parity])

    for gi in range(2):
        g, lo, hi, lo_al, m = params[gi]
        n = hi - lo

        def pair_body(c2, carry, lo=lo, hi=hi, lo_al=lo_al, m=m, gi=gi):
            for parity in range(2):
                c = c2 * 2 + parity
                bf = bufs[gi][parity]
                w = _win(lo_al, c)
                pltpu.make_async_copy(
                    h_hbm.at[pl.ds(w, CHUNK)], bf, sems[gi][parity]).wait()
                base = lo_al + c * CHUNK
                r_start = jnp.clip(jnp.maximum(lo, base) - w, 0, CHUNK)
                r_end = jnp.clip(jnp.minimum(hi, base + CHUNK) - w, 0, CHUNK)
                r_end = jnp.maximum(r_start, r_end)
                nu = (r_end - r_start) // UNROLL

                def rows_at(r0, k, carry2, bf=bf):
                    sums2, maxs2 = carry2
                    new_s = list(sums2)
                    new_m = list(maxs2)
                    for j in range(NCOLG):
                        for rr in range(k):
                            v = bf[r0 + rr, pl.ds(j * LANES, LANES)]
                            new_s[j] = new_s[j] + v
                            new_m[j] = jnp.maximum(new_m[j], v)
                    return (tuple(new_s), tuple(new_m))

                def blk_body(i, carry2, rs=r_start):
                    return rows_at(rs + i * UNROLL, UNROLL, carry2)

                def row_body(r, carry2):
                    return rows_at(r, 1, carry2)

                carry = lax.fori_loop(0, nu, blk_body, carry)
                carry = lax.fori_loop(r_start + nu * UNROLL, r_end, row_body,
                                      carry)

                @pl.when(c + 2 < m)
                def _(c=c, bf=bf, parity=parity, lo_al=lo_al, gi=gi):
                    pltpu.async_copy(h_hbm.at[pl.ds(_win(lo_al, c + 2), CHUNK)],
                                     bf, sems[gi][parity])
            return carry

        init = (tuple(jnp.zeros((LANES,), jnp.float32) for _ in range(NCOLG)),
                tuple(jnp.full((LANES,), -jnp.inf, jnp.float32)
                      for _ in range(NCOLG)))
        sums, maxs = lax.fori_loop(0, m // 2, pair_body, init)

        for j in range(NCOLG):
            sum_st[gi, pl.ds(j * LANES, LANES)] = sums[j]
            max_st[gi, pl.ds(j * LANES, LANES)] = maxs[j]
        cnt_st[gi, pl.ds(0, LANES)] = (jnp.full((LANES,), 1.0, jnp.float32)
                                       * n.astype(jnp.float32))
        pltpu.async_copy(sum_st.at[pl.ds(gi, 1)], sum_hbm.at[pl.ds(g, 1)], osem)
        pltpu.async_copy(max_st.at[pl.ds(gi, 1)], max_hbm.at[pl.ds(g, 1)], osem)
        pltpu.async_copy(cnt_st.at[pl.ds(gi, 1)], cnt_hbm.at[pl.ds(g, 1)], osem)

    for gi in range(2):
        g = params[gi][0]
        pltpu.make_async_copy(sum_st.at[pl.ds(gi, 1)],
                              sum_hbm.at[pl.ds(g, 1)], osem).wait()
        pltpu.make_async_copy(max_st.at[pl.ds(gi, 1)],
                              max_hbm.at[pl.ds(g, 1)], osem).wait()
        pltpu.make_async_copy(cnt_st.at[pl.ds(gi, 1)],
                              cnt_hbm.at[pl.ds(g, 1)], osem).wait()


def _sc_pool(h, starts128):
    mesh = plsc.VectorSubcoreMesh(core_axis_name="c", subcore_axis_name="s")
    f = pl.kernel(
        _sc_pool_body,
        mesh=mesh,
        out_type=[
            jax.ShapeDtypeStruct((NUM_GRAPHS, D), jnp.float32),
            jax.ShapeDtypeStruct((NUM_GRAPHS, D), jnp.float32),
            jax.ShapeDtypeStruct((NUM_GRAPHS, LANES), jnp.float32),
        ],
        scratch_types=[
            pltpu.VMEM((1, 128), jnp.int32),
            pltpu.VMEM((CHUNK, D), jnp.float32),
            pltpu.VMEM((CHUNK, D), jnp.float32),
            pltpu.VMEM((CHUNK, D), jnp.float32),
            pltpu.VMEM((CHUNK, D), jnp.float32),
            pltpu.VMEM((2, D), jnp.float32),
            pltpu.VMEM((2, D), jnp.float32),
            pltpu.VMEM((2, LANES), jnp.float32),
            pltpu.SemaphoreType.DMA,
            pltpu.SemaphoreType.DMA,
            pltpu.SemaphoreType.DMA,
            pltpu.SemaphoreType.DMA,
            pltpu.SemaphoreType.DMA,
        ],
    )
    return f(h, starts128)


def _mlp_body(cnt_ref, sum_ref, max_ref, w1_ref, b1_ref, w2_ref, b2_ref,
              out_ref):
    cnt = jnp.maximum(cnt_ref[:, 0:1], 1.0)
    h_sum = sum_ref[...]
    h_max = max_ref[...]
    h_mean = h_sum / cnt
    x = jnp.concatenate([h_sum, h_mean, h_max], axis=1)
    y = lax.dot_general(x, w1_ref[...], (((1,), (1,)), ((), ())),
                        preferred_element_type=jnp.float32) + b1_ref[...]
    y = 0.5 * y * (1.0 + lax.erf(y * 0.7071067811865476))
    out_ref[...] = lax.dot_general(y, w2_ref[...], (((1,), (1,)), ((), ())),
                                   preferred_element_type=jnp.float32) + b2_ref[...]


def _mlp(cnt, h_sum, h_max, W1, b1, W2, b2):
    return pl.pallas_call(
        _mlp_body,
        out_shape=jax.ShapeDtypeStruct((NUM_GRAPHS, D), jnp.float32),
    )(cnt, h_sum, h_max, W1, b1.reshape(1, D), W2, b2.reshape(1, D))


def kernel(h, batch, W1, b1, W2, b2):
    starts128 = _starts(batch)
    h_sum, h_max, cnt = _sc_pool(h, starts128)
    return _mlp(cnt, h_sum, h_max, W1, b1, W2, b2)


# speedup vs baseline: 1.1916x; 1.0295x over previous
import functools

import jax
import jax.numpy as jnp
from jax import lax
from jax.experimental import pallas as pl
from jax.experimental.pallas import tpu as pltpu
from jax.experimental.pallas import tpu_sc as plsc

N = 50000
D = 256
NUM_GRAPHS = 64
CHUNK = 120
LANES = 16
NCOLG = D // LANES
N_PAD = 50048
UNROLL = 8


def _scalar_at(ref, idx):
    v = ref[0, pl.ds(idx, LANES)]
    return v[0]


def _starts_body(batch_ref, starts_ref):
    b = batch_ref[...]
    lane = lax.broadcasted_iota(jnp.int32, (1, 128), 1)
    acc = jnp.zeros((1, 128), jnp.int32)
    for g in range(NUM_GRAPHS + 1):
        c = jnp.sum((b < g).astype(jnp.int32))
        acc = acc + jnp.where(lane == g, c, 0)
    starts_ref[...] = acc


def _starts(batch):
    batch_p = jnp.concatenate(
        [batch.astype(jnp.int32),
         jnp.full((N_PAD - N,), NUM_GRAPHS, jnp.int32)]).reshape(
             N_PAD // 128, 128)
    return pl.pallas_call(
        _starts_body,
        out_shape=jax.ShapeDtypeStruct((1, 128), jnp.int32),
    )(batch_p)


def _win(lo_al, c):
    return pl.multiple_of(jnp.minimum(lo_al + c * CHUNK, N - CHUNK), 8)


def _sc_pool_body(h_hbm, starts_hbm, sum_hbm, max_hbm, cnt_hbm,
                  starts_v, buf0, buf1, buf2, buf3, sum_st, max_st, cnt_st,
                  sem0, sem1, sem2, sem3, osem):
    wid = lax.axis_index("s") * 2 + lax.axis_index("c")
    pltpu.sync_copy(starts_hbm, starts_v)
    bufs = ((buf0, buf1), (buf2, buf3))
    sems = ((sem0, sem1), (sem2, sem3))

    params = []
    for gi in range(2):
        g = wid * 2 + gi
        lo = _scalar_at(starts_v, g)
        hi = _scalar_at(starts_v, g + 1)
        lo_al = (lo // 8) * 8
        num_chunks = jnp.maximum((hi - lo_al + CHUNK - 1) // CHUNK, 1)
        m = ((num_chunks + 1) // 2) * 2
        params.append((g, lo, hi, lo_al, m))
        for parity in range(2):
            pltpu.async_copy(h_hbm.at[pl.ds(_win(lo_al, parity), CHUNK)],
                             bufs[gi][parity], sems[gi][parity])

    for gi in range(2):
        g, lo, hi, lo_al, m = params[gi]
        n = hi - lo

        def pair_body(c2, carry, lo=lo, hi=hi, lo_al=lo_al, m=m, gi=gi):
            for parity in range(2):
                c = c2 * 2 + parity
                bf = bufs[gi][parity]
                w = _win(lo_al, c)
                pltpu.make_async_copy(
                    h_hbm.at[pl.ds(w, CHUNK)], bf, sems[gi][parity]).wait()
                base = lo_al + c * CHUNK
                r_start = jnp.clip(jnp.maximum(lo, base) - w, 0, CHUNK)
                r_end = jnp.clip(jnp.minimum(hi, base + CHUNK) - w, 0, CHUNK)
                r_end = jnp.maximum(r_start, r_end)
                nu = (r_end - r_start) // UNROLL

                def rows_at(r0, k, carry2, bf=bf):
                    sums2, maxs2 = carry2
                    new_s = list(sums2)
                    new_m = list(maxs2)
                    for j in range(NCOLG):
                        for rr in range(k):
                            v = bf[r0 + rr, pl.ds(j * LANES, LANES)]
                            new_s[j] = new_s[j] + v
                            new_m[j] = jnp.maximum(new_m[j], v)
                    return (tuple(new_s), tuple(new_m))

                def blk_body(i, carry2, rs=r_start):
                    return rows_at(rs + i * UNROLL, UNROLL, carry2)

                def row_body(r, carry2):
                    return rows_at(r, 1, carry2)

                carry = lax.fori_loop(0, nu, blk_body, carry)
                carry = lax.fori_loop(r_start + nu * UNROLL, r_end, row_body,
                                      carry)

                @pl.when(c + 2 < m)
                def _(c=c, bf=bf, parity=parity, lo_al=lo_al, gi=gi):
                    pltpu.async_copy(h_hbm.at[pl.ds(_win(lo_al, c + 2), CHUNK)],
                                     bf, sems[gi][parity])
            return carry

        init = (tuple(jnp.zeros((LANES,), jnp.float32) for _ in range(NCOLG)),
                tuple(jnp.full((LANES,), -jnp.inf, jnp.float32)
                      for _ in range(NCOLG)))
        sums, maxs = lax.fori_loop(0, m // 2, pair_body, init)

        for j in range(NCOLG):
            sum_st[gi, pl.ds(j * LANES, LANES)] = sums[j]
            max_st[gi, pl.ds(j * LANES, LANES)] = maxs[j]
        cnt_st[gi, pl.ds(0, LANES)] = (jnp.full((LANES,), 1.0, jnp.float32)
                                       * n.astype(jnp.float32))
        pltpu.async_copy(sum_st.at[pl.ds(gi, 1)], sum_hbm.at[pl.ds(g, 1)], osem)
        pltpu.async_copy(max_st.at[pl.ds(gi, 1)], max_hbm.at[pl.ds(g, 1)], osem)
        pltpu.async_copy(cnt_st.at[pl.ds(gi, 1)], cnt_hbm.at[pl.ds(g, 1)], osem)

    for gi in range(2):
        g = params[gi][0]
        pltpu.make_async_copy(sum_st.at[pl.ds(gi, 1)],
                              sum_hbm.at[pl.ds(g, 1)], osem).wait()
        pltpu.make_async_copy(max_st.at[pl.ds(gi, 1)],
                              max_hbm.at[pl.ds(g, 1)], osem).wait()
        pltpu.make_async_copy(cnt_st.at[pl.ds(gi, 1)],
                              cnt_hbm.at[pl.ds(g, 1)], osem).wait()


def _sc_pool(h, starts128):
    mesh = plsc.VectorSubcoreMesh(core_axis_name="c", subcore_axis_name="s")
    f = pl.kernel(
        _sc_pool_body,
        mesh=mesh,
        out_type=[
            jax.ShapeDtypeStruct((NUM_GRAPHS, D), jnp.float32),
            jax.ShapeDtypeStruct((NUM_GRAPHS, D), jnp.float32),
            jax.ShapeDtypeStruct((NUM_GRAPHS, LANES), jnp.float32),
        ],
        scratch_types=[
            pltpu.VMEM((1, 128), jnp.int32),
            pltpu.VMEM((CHUNK, D), jnp.float32),
            pltpu.VMEM((CHUNK, D), jnp.float32),
            pltpu.VMEM((CHUNK, D), jnp.float32),
            pltpu.VMEM((CHUNK, D), jnp.float32),
            pltpu.VMEM((2, D), jnp.float32),
            pltpu.VMEM((2, D), jnp.float32),
            pltpu.VMEM((2, LANES), jnp.float32),
            pltpu.SemaphoreType.DMA,
            pltpu.SemaphoreType.DMA,
            pltpu.SemaphoreType.DMA,
            pltpu.SemaphoreType.DMA,
            pltpu.SemaphoreType.DMA,
        ],
    )
    return f(h, starts128)


def _mlp_body(cnt_ref, sum_ref, max_ref, w1_ref, b1_ref, w2_ref, b2_ref,
              out_ref):
    cnt = jnp.maximum(cnt_ref[:, 0:1], 1.0)
    h_sum = sum_ref[...]
    h_max = max_ref[...]
    h_mean = h_sum / cnt
    x = jnp.concatenate([h_sum, h_mean, h_max], axis=1)
    y = lax.dot_general(x, w1_ref[...], (((1,), (1,)), ((), ())),
                        preferred_element_type=jnp.float32) + b1_ref[...]
    y = 0.5 * y * (1.0 + lax.erf(y * 0.7071067811865476))
    out_ref[...] = lax.dot_general(y, w2_ref[...], (((1,), (1,)), ((), ())),
                                   preferred_element_type=jnp.float32) + b2_ref[...]


def _mlp(cnt, h_sum, h_max, W1, b1, W2, b2):
    return pl.pallas_call(
        _mlp_body,
        out_shape=jax.ShapeDtypeStruct((NUM_GRAPHS, D), jnp.float32),
    )(cnt, h_sum, h_max, W1, b1.reshape(1, D), W2, b2.reshape(1, D))


def kernel(h, batch, W1, b1, W2, b2):
    starts128 = _starts(batch)
    h_sum, h_max, cnt = _sc_pool(h, starts128)
    return _mlp(cnt, h_sum, h_max, W1, b1, W2, b2)
